# baseline (speedup 1.0000x reference)
import jax
import jax.numpy as jnp
from jax.experimental import pallas as pl

_INV_T = 20.0
_TILE_N = 2048


def _mm_kernel(x_ref, m_ref, o_ref):
    a = x_ref[...] * _INV_T
    o_ref[...] = jax.lax.dot_general(
        a, m_ref[...],
        dimension_numbers=(((1,), (1,)), ((), ())),
        preferred_element_type=jnp.float32)


def kernel(x, y, memory):
    del y
    b, k = x.shape
    n = memory.shape[0]
    return pl.pallas_call(
        _mm_kernel,
        grid=(pl.cdiv(n, _TILE_N),),
        in_specs=[
            pl.BlockSpec((b, k), lambda i: (0, 0)),
            pl.BlockSpec((_TILE_N, k), lambda i: (i, 0)),
        ],
        out_specs=pl.BlockSpec((b, _TILE_N), lambda i: (0, i)),
        out_shape=jax.ShapeDtypeStruct((b, n), jnp.float32),
    )(x, memory)
